# baseline (speedup 1.0000x reference)
import jax
import jax.numpy as jnp
from jax.experimental import pallas as pl
from jax.experimental.pallas import tpu as pltpu

_EPS = 1e-07
_KT = 2048
_KT_PAIR = 1024


def _vq_levels(x0, refs, iota0, kt):
    T = x0.shape[1]
    r = x0
    rate = jnp.zeros((1, 1), jnp.float32)
    for cb_ref, cbt_ref, cbn_ref in refs:
        K = cb_ref.shape[1]
        KT = min(K, kt)
        nt = K // KT
        io = iota0[:KT] if KT < kt else iota0
        M = S = A = None
        r2 = r + r
        for t in range(nt):
            cbt = cb_ref[0, t * KT:(t + 1) * KT, :]
            dots = jax.lax.dot_general(
                cbt, r2, (((1,), (0,)), ((), ())),
                preferred_element_type=jnp.float32)
            tt = dots - cbn_ref[0, t * KT:(t + 1) * KT, :]
            tmax = jnp.max(tt, axis=0, keepdims=True)
            targ = jnp.min(jnp.where(tt == tmax, io, jnp.int32(K)),
                           axis=0, keepdims=True) + t * KT
            tsum = jnp.sum(jnp.exp(tt - tmax), axis=0, keepdims=True)
            if t == 0:
                M, S, A = tmax, tsum, targ
            else:
                better = tmax > M
                Mn = jnp.maximum(M, tmax)
                S = S * jnp.exp(M - Mn) + tsum * jnp.exp(tmax - Mn)
                A = jnp.where(better, targ, A)
                M = Mn
        sel = None
        for t in range(nt):
            c0 = cbt_ref[0, :, t * KT:(t + 1) * KT]
            hi = c0.astype(jnp.bfloat16)
            r1 = c0 - hi.astype(jnp.float32)
            mid = r1.astype(jnp.bfloat16)
            lo = (r1 - mid.astype(jnp.float32)).astype(jnp.bfloat16)
            oh = (io == (A - t * KT)).astype(jnp.bfloat16)
            g = None
            for part in (hi, mid, lo):
                gp = jax.lax.dot_general(
                    part, oh, (((1,), (0,)), ((), ())),
                    preferred_element_type=jnp.float32)
                g = gp if g is None else g + gp
            sel = g if sel is None else sel + g
        r = r - sel
        rate = rate + jnp.sum(-jnp.log(1.0 / S + _EPS), keepdims=True)
    return x0 - r, rate


def _vq_group_body(x_ref, cb0_ref, cb1_ref, cb2_ref,
                   cbt0_ref, cbt1_ref, cbt2_ref,
                   cbn0_ref, cbn1_ref, cbn2_ref,
                   q_ref, rate_ref):
    T = x_ref.shape[2]
    iota0 = jax.lax.broadcasted_iota(jnp.int32, (_KT, T), 0)
    q, rate = _vq_levels(x_ref[0],
                         ((cb0_ref, cbt0_ref, cbn0_ref),
                          (cb1_ref, cbt1_ref, cbn1_ref),
                          (cb2_ref, cbt2_ref, cbn2_ref)), iota0, _KT)
    q_ref[0] = q
    rate_ref[0] = rate


def _vq_pair_body(xa_ref, xb_ref,
                  cba0_ref, cba1_ref, cba2_ref,
                  cbta0_ref, cbta1_ref, cbta2_ref,
                  cbna0_ref, cbna1_ref, cbna2_ref,
                  cbb0_ref, cbb1_ref, cbb2_ref,
                  cbtb0_ref, cbtb1_ref, cbtb2_ref,
                  cbnb0_ref, cbnb1_ref, cbnb2_ref,
                  qa_ref, qb_ref, ratea_ref, rateb_ref):
    T = xa_ref.shape[2]
    iota0 = jax.lax.broadcasted_iota(jnp.int32, (_KT_PAIR, T), 0)
    qa, ra = _vq_levels(xa_ref[0],
                        ((cba0_ref, cbta0_ref, cbna0_ref),
                         (cba1_ref, cbta1_ref, cbna1_ref),
                         (cba2_ref, cbta2_ref, cbna2_ref)), iota0, _KT_PAIR)
    qa_ref[0] = qa
    ratea_ref[0] = ra
    qb, rb = _vq_levels(xb_ref[0] - qa,
                        ((cbb0_ref, cbtb0_ref, cbnb0_ref),
                         (cbb1_ref, cbtb1_ref, cbnb1_ref),
                         (cbb2_ref, cbtb2_ref, cbnb2_ref)), iota0, _KT_PAIR)
    qb_ref[0] = qb
    rateb_ref[0] = rb


def _vq_group(x, cb0, cb1, cb2):
    m, d, T = x.shape
    cbts = [jnp.transpose(cb, (0, 2, 1)) for cb in (cb0, cb1, cb2)]
    cbns = [jnp.sum(cb * cb, axis=-1)[..., None] for cb in (cb0, cb1, cb2)]
    q, rate = pl.pallas_call(
        _vq_group_body,
        grid=(m,),
        in_specs=[
            pl.BlockSpec((1, d, T), lambda i: (i, 0, 0)),
            *[pl.BlockSpec((1, cb.shape[1], d), lambda i: (i, 0, 0))
              for cb in (cb0, cb1, cb2)],
            *[pl.BlockSpec((1, d, cb.shape[1]), lambda i: (i, 0, 0))
              for cb in (cb0, cb1, cb2)],
            *[pl.BlockSpec((1, cb.shape[1], 1), lambda i: (i, 0, 0))
              for cb in (cb0, cb1, cb2)],
        ],
        out_specs=[
            pl.BlockSpec((1, d, T), lambda i: (i, 0, 0)),
            pl.BlockSpec((1, 1, 1), lambda i: (i, 0, 0)),
        ],
        out_shape=[
            jax.ShapeDtypeStruct((m, d, T), jnp.float32),
            jax.ShapeDtypeStruct((m, 1, 1), jnp.float32),
        ],
        compiler_params=pltpu.CompilerParams(
            dimension_semantics=("arbitrary",)),
    )(x, cb0, cb1, cb2, *cbts, *cbns)
    return q, jnp.sum(rate) / jnp.float32(T * m)


def _vq_pair(xa, xb, cbsa, cbsb):
    m, d, T = xa.shape
    cball = list(cbsa) + list(cbsb)
    cbts = [jnp.transpose(cb, (0, 2, 1)) for cb in cball]
    cbns = [jnp.sum(cb * cb, axis=-1)[..., None] for cb in cball]
    args = [xa, xb,
            cbsa[0], cbsa[1], cbsa[2], *cbts[:3], *cbns[:3],
            cbsb[0], cbsb[1], cbsb[2], *cbts[3:], *cbns[3:]]
    specs = [pl.BlockSpec((1, d, T), lambda i: (i, 0, 0)),
             pl.BlockSpec((1, d, T), lambda i: (i, 0, 0))]
    for cbs in (cbsa, cbsb):
        specs += [pl.BlockSpec((1, cb.shape[1], d), lambda i: (i, 0, 0))
                  for cb in cbs]
        specs += [pl.BlockSpec((1, d, cb.shape[1]), lambda i: (i, 0, 0))
                  for cb in cbs]
        specs += [pl.BlockSpec((1, cb.shape[1], 1), lambda i: (i, 0, 0))
                  for cb in cbs]
    qa, qb, ra, rb = pl.pallas_call(
        _vq_pair_body,
        grid=(m,),
        in_specs=specs,
        out_specs=[
            pl.BlockSpec((1, d, T), lambda i: (i, 0, 0)),
            pl.BlockSpec((1, d, T), lambda i: (i, 0, 0)),
            pl.BlockSpec((1, 1, 1), lambda i: (i, 0, 0)),
            pl.BlockSpec((1, 1, 1), lambda i: (i, 0, 0)),
        ],
        out_shape=[
            jax.ShapeDtypeStruct((m, d, T), jnp.float32),
            jax.ShapeDtypeStruct((m, d, T), jnp.float32),
            jax.ShapeDtypeStruct((m, 1, 1), jnp.float32),
            jax.ShapeDtypeStruct((m, 1, 1), jnp.float32),
        ],
        compiler_params=pltpu.CompilerParams(
            dimension_semantics=("arbitrary",)),
    )(*args)
    return (qa, qb,
            jnp.sum(ra) / jnp.float32(T * m),
            jnp.sum(rb) / jnp.float32(T * m))


def _to_tokens(x, m):
    B, c, H, W = x.shape
    d = c // m
    return x.reshape(B, m, d, H * W).transpose(1, 2, 0, 3).reshape(m, d, B * H * W)


def _from_tokens(q, B, c, H, W, m):
    d = c // m
    return q.reshape(m, d, B, H * W).transpose(2, 0, 1, 3).reshape(B, c, H, W)


def kernel(y0, y1, y2, y3, y4,
           cb0_0, cb0_1, cb0_2,
           cb1_0, cb1_1, cb1_2,
           cb2_0, cb2_1, cb2_2,
           cb3_0, cb3_1, cb3_2,
           cb4_0, cb4_1, cb4_2):
    ys = [y0, y1, y2, y3, y4]
    cbs = [[cb0_0, cb0_1, cb0_2],
           [cb1_0, cb1_1, cb1_2],
           [cb2_0, cb2_1, cb2_2],
           [cb3_0, cb3_1, cb3_2],
           [cb4_0, cb4_1, cb4_2]]
    B, _, H, W = y0.shape
    q0t, q1t, rate0, rate1 = _vq_pair(
        _to_tokens(y0, 4), _to_tokens(y1, 4), cbs[0], cbs[1])
    q0 = _from_tokens(q0t, B, 16, H, W, 4)
    q1 = _from_tokens(q1t, B, 16, H, W, 4)
    dec = [q0, q1 + q0]
    rates = [rate0, rate1]
    f = jnp.concatenate([q0, q1], axis=1)
    for i in range(2, 5):
        m = cbs[i][0].shape[0]
        c = ys[i].shape[1]
        nin = ys[i] - f
        xt = _to_tokens(nin, m)
        qt, rate = _vq_group(xt, *cbs[i])
        q = _from_tokens(qt, B, c, H, W, m)
        dec.append(q + f)
        f = jnp.concatenate([f, q], axis=1)
        rates.append(rate)
    return jnp.concatenate(dec, axis=1), jnp.stack(rates)


# speedup vs baseline: 1.1054x; 1.1054x over previous
import jax
import jax.numpy as jnp
from jax.experimental import pallas as pl
from jax.experimental.pallas import tpu as pltpu

_EPS = 1e-07
_KT = 2048


def _vq_group_body(x_ref, cb0_ref, cb1_ref, cb2_ref,
                   cbt0_ref, cbt1_ref, cbt2_ref,
                   cbn0_ref, cbn1_ref, cbn2_ref,
                   q_ref, rate_ref):
    x0 = x_ref[0]
    T = x0.shape[1]
    r = x0
    rate = jnp.zeros((1, 1), jnp.float32)
    iota0 = jax.lax.broadcasted_iota(jnp.int32, (_KT, T), 0)
    for cb_ref, cbt_ref, cbn_ref in ((cb0_ref, cbt0_ref, cbn0_ref),
                                     (cb1_ref, cbt1_ref, cbn1_ref),
                                     (cb2_ref, cbt2_ref, cbn2_ref)):
        K = cb_ref.shape[1]
        KT = min(K, _KT)
        nt = K // KT
        io = iota0[:KT] if KT < _KT else iota0
        M = S = A = None
        r2 = r + r
        for t in range(nt):
            cbt = cb_ref[0, t * KT:(t + 1) * KT, :]
            dots = jax.lax.dot_general(
                cbt, r2, (((1,), (0,)), ((), ())),
                preferred_element_type=jnp.float32)
            tt = dots - cbn_ref[0, t * KT:(t + 1) * KT, :]
            tmax = jnp.max(tt, axis=0, keepdims=True)
            targ = jnp.min(jnp.where(tt == tmax, io, jnp.int32(K)),
                           axis=0, keepdims=True) + t * KT
            tsum = jnp.sum(jnp.exp(tt - tmax), axis=0, keepdims=True)
            if t == 0:
                M, S, A = tmax, tsum, targ
            else:
                better = tmax > M
                Mn = jnp.maximum(M, tmax)
                S = S * jnp.exp(M - Mn) + tsum * jnp.exp(tmax - Mn)
                A = jnp.where(better, targ, A)
                M = Mn
        sel = None
        for t in range(nt):
            c0 = cbt_ref[0, :, t * KT:(t + 1) * KT]
            hi = c0.astype(jnp.bfloat16)
            r1 = c0 - hi.astype(jnp.float32)
            mid = r1.astype(jnp.bfloat16)
            lo = (r1 - mid.astype(jnp.float32)).astype(jnp.bfloat16)
            oh = (io == (A - t * KT)).astype(jnp.bfloat16)
            g = None
            for part in (hi, mid, lo):
                gp = jax.lax.dot_general(
                    part, oh, (((1,), (0,)), ((), ())),
                    preferred_element_type=jnp.float32)
                g = gp if g is None else g + gp
            sel = g if sel is None else sel + g
        r = r - sel
        rate = rate + jnp.sum(-jnp.log(1.0 / S + _EPS), keepdims=True)
    q_ref[0] = x0 - r
    rate_ref[0] = rate


def _vq_group(x, cb0, cb1, cb2):
    m, d, T = x.shape
    cbts = [jnp.transpose(cb, (0, 2, 1)) for cb in (cb0, cb1, cb2)]
    cbns = [jnp.sum(cb * cb, axis=-1)[..., None] for cb in (cb0, cb1, cb2)]
    q, rate = pl.pallas_call(
        _vq_group_body,
        grid=(m,),
        in_specs=[
            pl.BlockSpec((1, d, T), lambda i: (i, 0, 0)),
            *[pl.BlockSpec((1, cb.shape[1], d), lambda i: (i, 0, 0))
              for cb in (cb0, cb1, cb2)],
            *[pl.BlockSpec((1, d, cb.shape[1]), lambda i: (i, 0, 0))
              for cb in (cb0, cb1, cb2)],
            *[pl.BlockSpec((1, cb.shape[1], 1), lambda i: (i, 0, 0))
              for cb in (cb0, cb1, cb2)],
        ],
        out_specs=[
            pl.BlockSpec((1, d, T), lambda i: (i, 0, 0)),
            pl.BlockSpec((1, 1, 1), lambda i: (i, 0, 0)),
        ],
        out_shape=[
            jax.ShapeDtypeStruct((m, d, T), jnp.float32),
            jax.ShapeDtypeStruct((m, 1, 1), jnp.float32),
        ],
        compiler_params=pltpu.CompilerParams(
            dimension_semantics=("arbitrary",)),
    )(x, cb0, cb1, cb2, *cbts, *cbns)
    return q, jnp.sum(rate) / jnp.float32(T * m)


def _to_tokens(x, m):
    B, c, H, W = x.shape
    d = c // m
    return x.reshape(B, m, d, H * W).transpose(1, 2, 0, 3).reshape(m, d, B * H * W)


def _from_tokens(q, B, c, H, W, m):
    d = c // m
    return q.reshape(m, d, B, H * W).transpose(2, 0, 1, 3).reshape(B, c, H, W)


def kernel(y0, y1, y2, y3, y4,
           cb0_0, cb0_1, cb0_2,
           cb1_0, cb1_1, cb1_2,
           cb2_0, cb2_1, cb2_2,
           cb3_0, cb3_1, cb3_2,
           cb4_0, cb4_1, cb4_2):
    ys = [y0, y1, y2, y3, y4]
    cbs = [[cb0_0, cb0_1, cb0_2],
           [cb1_0, cb1_1, cb1_2],
           [cb2_0, cb2_1, cb2_2],
           [cb3_0, cb3_1, cb3_2],
           [cb4_0, cb4_1, cb4_2]]
    B, _, H, W = y0.shape
    dec = []
    rates = []
    f = None
    for i in range(5):
        m = cbs[i][0].shape[0]
        c = ys[i].shape[1]
        nin = ys[i] if f is None else ys[i] - f
        xt = _to_tokens(nin, m)
        qt, rate = _vq_group(xt, *cbs[i])
        q = _from_tokens(qt, B, c, H, W, m)
        dec.append(q if f is None else q + f)
        f = q if f is None else jnp.concatenate([f, q], axis=1)
        rates.append(rate)
    return jnp.concatenate(dec, axis=1), jnp.stack(rates)
